# baseline (speedup 1.0000x reference)
import functools

import jax
import jax.numpy as jnp
from jax import lax
from jax.experimental import pallas as pl
from jax.experimental.pallas import tpu as pltpu
from jax.experimental.pallas import tpu_sc as plsc

N_BUF = 3
LANES = 128


@functools.lru_cache(maxsize=None)
def _make_gather(batch: int, hist: int, dim: int):
    info = plsc.get_sparse_core_info()
    num_cores, num_subcores = info.num_cores, info.num_subcores
    num_workers = num_cores * num_subcores
    assert batch % num_workers == 0
    per_worker = batch // num_workers
    hist_pad = (hist + 7) // 8 * 8
    assert 2 * dim == LANES
    assert per_worker >= 6 and (per_worker - 5) % 3 == 0
    n_groups = (per_worker - 5) // 3

    mesh = plsc.VectorSubcoreMesh(core_axis_name="c", subcore_axis_name="s")

    @functools.partial(
        pl.kernel,
        mesh=mesh,
        out_type=jax.ShapeDtypeStruct((batch, hist_pad, LANES), jnp.float32),
        scratch_types=[
            pltpu.VMEM((per_worker, LANES), jnp.int32),
            [pltpu.VMEM((hist_pad, LANES), jnp.float32)] * N_BUF,
            [pltpu.SemaphoreType.DMA] * N_BUF,
            [pltpu.SemaphoreType.DMA] * N_BUF,
        ],
    )
    def gather(idx_hbm, tab_hbm, out_hbm, idx_v, bufs, sem_g, sem_w):
        wid = lax.axis_index("s") * num_cores + lax.axis_index("c")
        base_w = pl.multiple_of(wid * per_worker, 8)
        pltpu.sync_copy(idx_hbm.at[pl.ds(base_w, per_worker)], idx_v)

        def start_g(j, b):
            idx = idx_v.at[j, pl.ds(0, hist)]
            pltpu.async_copy(tab_hbm.at[idx], bufs[b].at[pl.ds(0, hist)],
                             sem_g[b])

        def wait_g(b):
            idx = idx_v.at[0, pl.ds(0, hist)]
            pltpu.make_async_copy(tab_hbm.at[idx],
                                  bufs[b].at[pl.ds(0, hist)], sem_g[b]).wait()

        def start_w(j, b):
            pltpu.async_copy(bufs[b], out_hbm.at[base_w + j], sem_w[b])

        def wait_w(b):
            pltpu.make_async_copy(bufs[b], out_hbm.at[0], sem_w[b]).wait()

        start_g(0, 0)
        start_g(1, 1)
        wait_g(0)
        start_w(0, 0)
        start_g(2, 2)
        wait_g(1)
        start_w(1, 1)
        wait_w(0)
        start_g(3, 0)
        wait_g(2)
        start_w(2, 2)
        wait_w(1)
        start_g(4, 1)
        wait_g(0)
        start_w(3, 0)

        def body(g, carry):
            for k in range(3):
                i = 4 + 3 * g + k
                b = (1 + k) % 3
                b_next = (2 + k) % 3
                wait_w(b_next)
                start_g(i + 1, b_next)
                wait_g(b)
                start_w(i, b)
            return carry

        lax.fori_loop(0, n_groups, body, 0)

        wait_g(1)
        start_w(per_worker - 1, 1)
        wait_w(2)
        wait_w(0)
        wait_w(1)

    return gather


@functools.lru_cache(maxsize=None)
def _make_split(batch: int, hist: int, dim: int, block: int):
    hist_pad = (hist + 7) // 8 * 8

    def split_body(fused_ref, amp_ref, ph_ref):
        x = fused_ref[...]
        amp_ref[...] = x[:, :hist, :dim]
        ph_ref[...] = x[:, :hist, dim:]

    return pl.pallas_call(
        split_body,
        grid=(batch // block,),
        in_specs=[
            pl.BlockSpec((block, hist_pad, LANES), lambda i: (i, 0, 0)),
        ],
        out_specs=(
            pl.BlockSpec((block, hist, dim), lambda i: (i, 0, 0)),
            pl.BlockSpec((block, hist, dim), lambda i: (i, 0, 0)),
        ),
        out_shape=(
            jax.ShapeDtypeStruct((batch, hist, dim), jnp.float32),
            jax.ShapeDtypeStruct((batch, hist, dim), jnp.float32),
        ),
        compiler_params=pltpu.CompilerParams(
            dimension_semantics=("arbitrary",),
        ),
    )


def kernel(indices, amplitude_table, phase_table):
    batch, hist = indices.shape
    dim = amplitude_table.shape[1]
    idx_pad = jnp.pad(indices, ((0, 0), (0, LANES - hist)))
    fused_table = jnp.concatenate([amplitude_table, phase_table], axis=1)
    fused = _make_gather(batch, hist, dim)(idx_pad, fused_table)
    amp, ph = _make_split(batch, hist, dim, 64)(fused)
    return amp, ph
